# baseline (speedup 1.0000x reference)
import functools

import jax
import jax.numpy as jnp
from jax import lax
from jax.experimental import pallas as pl
from jax.experimental.pallas import tpu as pltpu
from jax.experimental.pallas import tpu_sc as plsc

NUM_HEADS = 16
N = 256
TROW = 961
TABLE_WORDS = NUM_HEADS * TROW
NUM_WORKERS = 32
ROWS_PER_W = N // NUM_WORKERS
IDX_PER_W = ROWS_PER_W * N
BLK = N * NUM_HEADS
GROUPS_PER_ROW = N // 16


def _sc_gather(table_flat, idx_flat):
    mesh = plsc.VectorSubcoreMesh(core_axis_name="c", subcore_axis_name="s")

    @functools.partial(
        pl.kernel,
        mesh=mesh,
        out_type=jax.ShapeDtypeStruct((N * BLK,), jnp.float32),
        scratch_types=[
            pltpu.VMEM((TABLE_WORDS,), jnp.float32),
            pltpu.VMEM((IDX_PER_W,), jnp.int32),
            pltpu.VMEM((ROWS_PER_W * BLK,), jnp.float32),
            pltpu.SemaphoreType.DMA,
            pltpu.SemaphoreType.DMA,
        ],
        compiler_params=pltpu.CompilerParams(
            use_tc_tiling_on_sc=False, needs_layout_passes=False),
    )
    def gather_kernel(table_hbm, idx_hbm, out_hbm, table_v, idx_v, blk_v,
                      sem_in, sem_out):
        wid = lax.axis_index("s") * 2 + lax.axis_index("c")
        with jax.named_scope("stage_in"):
            half = TABLE_WORDS // 2
            cin = [
                pltpu.async_copy(idx_hbm.at[pl.ds(wid * IDX_PER_W, IDX_PER_W)],
                                 idx_v, sem_in)
            ]
            for k in range(2):
                off = ((wid + k) % 2) * half
                cin.append(pltpu.async_copy(table_hbm.at[pl.ds(off, half)],
                                            table_v.at[pl.ds(off, half)],
                                            sem_in))
            for c in cin:
                c.wait()

        with jax.named_scope("gather_loop"):
            def row_body(a, _):
                @plsc.parallel_loop(0, GROUPS_PER_ROW, unroll=2)
                def grp_body(j):
                    jb = (j >> 3) * 1024 + (j & 7) * 16
                    idx16 = idx_v[pl.ds(jb + a * 128, 16)]
                    base = a * BLK + jb
                    for c in range(NUM_HEADS):
                        vals = plsc.load_gather(table_v, [idx16 + c * TROW])
                        blk_v[pl.ds(base + (c >> 3) * 2048 + (c & 7) * 128,
                                    16)] = vals

                pltpu.async_copy(
                    blk_v.at[pl.ds(a * BLK, BLK)],
                    out_hbm.at[pl.ds((wid * ROWS_PER_W + a) * BLK, BLK)],
                    sem_out)
                return ()

            lax.fori_loop(0, ROWS_PER_W, row_body, (), unroll=False)
        with jax.named_scope("write_out"):
            pltpu.make_async_copy(
                out_hbm.at[pl.ds(wid * ROWS_PER_W * BLK, ROWS_PER_W * BLK)],
                blk_v, sem_out).wait()

    return gather_kernel(table_flat, idx_flat)


def kernel(relative_position_bias_table, relative_position_index):
    table_flat = relative_position_bias_table.T.reshape(-1)
    idx_flat = (relative_position_index.astype(jnp.int32)
                .reshape(32, 8, 2, 128)
                .transpose(0, 2, 1, 3)
                .reshape(-1))
    out = _sc_gather(table_flat, idx_flat)
    return (out.reshape(N, 2, 2, 8, 128)
               .transpose(0, 2, 4, 1, 3)
               .reshape(N, N, NUM_HEADS))


# speedup vs baseline: 1.0357x; 1.0357x over previous
import functools

import jax
import jax.numpy as jnp
from jax import lax
from jax.experimental import pallas as pl
from jax.experimental.pallas import tpu as pltpu
from jax.experimental.pallas import tpu_sc as plsc

NUM_HEADS = 16
N = 256
TROW = 968
TABLE_WORDS = NUM_HEADS * TROW
NUM_WORKERS = 32
ROWS_PER_W = N // NUM_WORKERS
IDX_PER_W = ROWS_PER_W * N
BLK = N * NUM_HEADS
GROUPS_PER_ROW = N // 16


def _sc_gather(table_flat, idx_flat):
    mesh = plsc.VectorSubcoreMesh(core_axis_name="c", subcore_axis_name="s")

    @functools.partial(
        pl.kernel,
        mesh=mesh,
        out_type=jax.ShapeDtypeStruct((N * BLK,), jnp.float32),
        scratch_types=[
            pltpu.VMEM((TABLE_WORDS,), jnp.float32),
            pltpu.VMEM((IDX_PER_W,), jnp.int32),
            pltpu.VMEM((ROWS_PER_W * BLK,), jnp.float32),
            pltpu.SemaphoreType.DMA,
            pltpu.SemaphoreType.DMA,
        ],
        compiler_params=pltpu.CompilerParams(
            use_tc_tiling_on_sc=False, needs_layout_passes=False),
    )
    def gather_kernel(table_hbm, idx_hbm, out_hbm, table_v, idx_v, blk_v,
                      sem_in, sem_out):
        wid = lax.axis_index("s") * 2 + lax.axis_index("c")
        qtr = TABLE_WORDS // 4
        cin = [
            pltpu.async_copy(idx_hbm.at[pl.ds(wid * IDX_PER_W, IDX_PER_W)],
                             idx_v, sem_in)
        ]
        for k in range(4):
            off = ((wid + k) % 4) * qtr
            cin.append(pltpu.async_copy(table_hbm.at[pl.ds(off, qtr)],
                                        table_v.at[pl.ds(off, qtr)],
                                        sem_in))
        for c in cin:
            c.wait()

        def row_body(a, _):
            @plsc.parallel_loop(0, GROUPS_PER_ROW, unroll=2)
            def grp_body(j):
                jb = (j >> 3) * 1024 + (j & 7) * 16
                idx16 = idx_v[pl.ds(jb + a * 128, 16)]
                base = a * BLK + jb
                for c in range(NUM_HEADS):
                    vals = plsc.load_gather(table_v, [idx16 + c * TROW])
                    blk_v[pl.ds(base + (c >> 3) * 2048 + (c & 7) * 128,
                                16)] = vals

            pltpu.async_copy(
                blk_v.at[pl.ds(a * BLK, BLK)],
                out_hbm.at[pl.ds((wid * ROWS_PER_W + a) * BLK, BLK)],
                sem_out)
            return ()

        lax.fori_loop(0, ROWS_PER_W, row_body, (), unroll=False)
        pltpu.make_async_copy(
            out_hbm.at[pl.ds(wid * ROWS_PER_W * BLK, ROWS_PER_W * BLK)],
            blk_v, sem_out).wait()

    return gather_kernel(table_flat, idx_flat)


def kernel(relative_position_bias_table, relative_position_index):
    table_flat = jnp.pad(relative_position_bias_table.T,
                         ((0, 0), (0, TROW - 961))).reshape(-1)
    idx_flat = (relative_position_index.astype(jnp.int32)
                .reshape(32, 8, 2, 128)
                .transpose(0, 2, 1, 3)
                .reshape(-1))
    out = _sc_gather(table_flat, idx_flat)
    return (out.reshape(N, 2, 2, 8, 128)
               .transpose(0, 2, 4, 1, 3)
               .reshape(N, N, NUM_HEADS))
